# baseline (device time: 12003 ns/iter reference)
import jax
import jax.numpy as jnp
from jax import lax
from jax.experimental import pallas as pl
from jax.experimental.pallas import tpu as pltpu

T = 256
V_LOCAL = 4096


def kernel(x, W, labels):
    def body(x_ref, w_ref, lab_ref, out_ref, send_ref, recv_ref,
             send_sem, recv_sem, send_sem2, recv_sem2):
        my_x = lax.axis_index("x")
        my_y = lax.axis_index("y")
        peer = (1 - my_x, my_y)

        barrier_sem = pltpu.get_barrier_semaphore()
        pl.semaphore_signal(barrier_sem, inc=1, device_id=peer,
                            device_id_type=pl.DeviceIdType.MESH)

        logits = jnp.dot(x_ref[:, :], w_ref[:, :],
                         preferred_element_type=jnp.float32)
        s = jnp.sum(jnp.exp(logits), axis=1)

        send_ref[0, :] = s
        pl.semaphore_wait(barrier_sem, 1)
        rdma_s = pltpu.make_async_remote_copy(
            src_ref=send_ref.at[0],
            dst_ref=recv_ref.at[0],
            send_sem=send_sem,
            recv_sem=recv_sem,
            device_id=peer,
            device_id_type=pl.DeviceIdType.MESH,
        )
        rdma_s.start()

        lab_local = lab_ref[:] - my_x * V_LOCAL
        col = lax.broadcasted_iota(jnp.int32, (T, V_LOCAL), 1)
        g = jnp.sum(jnp.where(col == lab_local[:, None], logits, 0.0),
                    axis=1)
        send_ref[1, :] = g
        rdma_g = pltpu.make_async_remote_copy(
            src_ref=send_ref.at[1],
            dst_ref=recv_ref.at[1],
            send_sem=send_sem2,
            recv_sem=recv_sem2,
            device_id=peer,
            device_id_type=pl.DeviceIdType.MESH,
        )
        rdma_g.start()

        rdma_s.wait_recv()
        lse = jnp.log(s + recv_ref[0, :])
        rdma_g.wait_recv()
        out_ref[:] = lse - (g + recv_ref[1, :])
        rdma_s.wait_send()
        rdma_g.wait_send()

    return pl.pallas_call(
        body,
        out_shape=jax.ShapeDtypeStruct((T,), jnp.float32),
        in_specs=[
            pl.BlockSpec(memory_space=pltpu.VMEM),
            pl.BlockSpec(memory_space=pltpu.VMEM),
            pl.BlockSpec(memory_space=pltpu.VMEM),
        ],
        out_specs=pl.BlockSpec(memory_space=pltpu.VMEM),
        scratch_shapes=[
            pltpu.VMEM((2, T), jnp.float32),
            pltpu.VMEM((2, T), jnp.float32),
            pltpu.SemaphoreType.DMA,
            pltpu.SemaphoreType.DMA,
            pltpu.SemaphoreType.DMA,
            pltpu.SemaphoreType.DMA,
        ],
        compiler_params=pltpu.CompilerParams(collective_id=0),
    )(x, W, labels)


# device time: 11658 ns/iter; 1.0296x vs baseline; 1.0296x over previous
import jax
import jax.numpy as jnp
from jax import lax
from jax.experimental import pallas as pl
from jax.experimental.pallas import tpu as pltpu

T = 256
V_LOCAL = 4096


def kernel(x, W, labels):
    def body(x_ref, w_ref, lab_ref, out_ref, send_ref, recv_ref,
             send_sem, recv_sem):
        my_x = lax.axis_index("x")
        my_y = lax.axis_index("y")
        peer = (1 - my_x, my_y)

        barrier_sem = pltpu.get_barrier_semaphore()
        pl.semaphore_signal(barrier_sem, inc=1, device_id=peer,
                            device_id_type=pl.DeviceIdType.MESH)

        logits = jnp.dot(x_ref[:, :], w_ref[:, :],
                         preferred_element_type=jnp.float32)
        s = jnp.sum(jnp.exp(logits), axis=1)
        lab_local = lab_ref[:] - my_x * V_LOCAL
        col = lax.broadcasted_iota(jnp.int32, (T, V_LOCAL), 1)
        g = jnp.sum(jnp.where(col == lab_local[:, None], logits, 0.0),
                    axis=1)

        send_ref[0, :] = s
        send_ref[1, :] = g

        pl.semaphore_wait(barrier_sem, 1)
        rdma = pltpu.make_async_remote_copy(
            src_ref=send_ref,
            dst_ref=recv_ref,
            send_sem=send_sem,
            recv_sem=recv_sem,
            device_id=peer,
            device_id_type=pl.DeviceIdType.MESH,
        )
        rdma.start()
        rdma.wait_recv()
        out_ref[:] = (jnp.log(s + recv_ref[0, :])
                      - (g + recv_ref[1, :]))
        rdma.wait_send()

    return pl.pallas_call(
        body,
        out_shape=jax.ShapeDtypeStruct((T,), jnp.float32),
        in_specs=[
            pl.BlockSpec(memory_space=pltpu.VMEM),
            pl.BlockSpec(memory_space=pltpu.VMEM),
            pl.BlockSpec(memory_space=pltpu.VMEM),
        ],
        out_specs=pl.BlockSpec(memory_space=pltpu.VMEM),
        scratch_shapes=[
            pltpu.VMEM((2, T), jnp.float32),
            pltpu.VMEM((2, T), jnp.float32),
            pltpu.SemaphoreType.DMA,
            pltpu.SemaphoreType.DMA,
        ],
        compiler_params=pltpu.CompilerParams(collective_id=0),
    )(x, W, labels)
